# baseline (speedup 1.0000x reference)
import jax
import jax.numpy as jnp
from jax import lax
from jax.experimental import pallas as pl
from jax.experimental.pallas import tpu as pltpu
from jax.experimental.pallas import tpu_sc as plsc

_N = 10000
_E = 320000
_D = 128
_NC = 2
_NS = 16
_NW = _NC * _NS
_CH = 128
_NCH = 80
_EPT = _NCH * _CH
_EPAD = _NW * _EPT
_NPAD = 10240
_RPD = _NPAD // _NS

_mesh = plsc.VectorSubcoreMesh(core_axis_name="c", subcore_axis_name="s")


def _deg_body(dst_hbm, ones_hbm, zeros_hbm, out_hbm, dst_v, ones_v, acc_sh, sem):
    del sem
    cid = lax.axis_index("c")
    sid = lax.axis_index("s")
    wid = sid * _NC + cid
    pltpu.sync_copy(zeros_hbm, acc_sh.at[pl.ds(sid * _RPD, _RPD)])
    pltpu.sync_copy(ones_hbm, ones_v)
    pltpu.sync_copy(dst_hbm.at[wid], dst_v)
    plsc.subcore_barrier()

    def body(j, carry):
        pltpu.sync_copy(ones_v, acc_sh.at[dst_v.at[j]], add=True)
        return carry

    lax.fori_loop(0, _NCH, body, 0)
    plsc.subcore_barrier()
    pltpu.sync_copy(acc_sh.at[pl.ds(sid * _RPD, _RPD)],
                    out_hbm.at[cid, pl.ds(sid * _RPD, _RPD)])


_deg_call = pl.kernel(
    _deg_body,
    out_type=jax.ShapeDtypeStruct((_NC, _NPAD), jnp.float32),
    mesh=_mesh,
    scratch_types=[
        pltpu.VMEM((_NCH, _CH), jnp.int32),
        pltpu.VMEM((_CH,), jnp.float32),
        pltpu.VMEM_SHARED((_NPAD,), jnp.float32),
        pltpu.SemaphoreType.DMA,
    ],
)


def _agg_body(hs_hbm, src_hbm, dst_hbm, zrows_hbm, out_hbm,
              src_v, dring, rows0, rows1, acc_sh, gsem0, gsem1, dsem0, dsem1):
    cid = lax.axis_index("c")
    sid = lax.axis_index("s")
    wid = sid * _NC + cid
    pltpu.sync_copy(src_hbm.at[wid], src_v)
    pltpu.async_copy(dst_hbm.at[wid, 0], dring.at[0], dsem0)
    pltpu.async_copy(dst_hbm.at[wid, 1], dring.at[1], dsem1)
    pltpu.async_copy(hs_hbm.at[src_v.at[0]], rows0, gsem0)
    pltpu.async_copy(hs_hbm.at[src_v.at[1]], rows1, gsem1)
    pltpu.sync_copy(zrows_hbm, acc_sh.at[pl.ds(sid * _RPD, _RPD)])
    plsc.subcore_barrier()

    def body(j2, carry):
        j = 2 * j2
        for b, buf, gsem, dsem in ((0, rows0, gsem0, dsem0),
                                   (1, rows1, gsem1, dsem1)):
            jj = j + b
            pltpu.make_async_copy(hs_hbm.at[src_v.at[jj]], buf, gsem).wait()
            pltpu.make_async_copy(dst_hbm.at[wid, jj], dring.at[b], dsem).wait()
            pltpu.sync_copy(buf, acc_sh.at[dring.at[b]], add=True)

            @pl.when(jj + 2 < _NCH)
            def _():
                pltpu.async_copy(dst_hbm.at[wid, jj + 2], dring.at[b], dsem)
                pltpu.async_copy(hs_hbm.at[src_v.at[jj + 2]], buf, gsem)

        return carry

    lax.fori_loop(0, _NCH // 2, body, 0)
    plsc.subcore_barrier()
    pltpu.sync_copy(acc_sh.at[pl.ds(sid * _RPD, _RPD)],
                    out_hbm.at[cid, pl.ds(sid * _RPD, _RPD)])


_agg_call = pl.kernel(
    _agg_body,
    out_type=jax.ShapeDtypeStruct((_NC, _NPAD, _D), jnp.float32),
    mesh=_mesh,
    scratch_types=[
        pltpu.VMEM((_NCH, _CH), jnp.int32),
        pltpu.VMEM((2, _CH), jnp.int32),
        pltpu.VMEM((_CH, _D), jnp.float32),
        pltpu.VMEM((_CH, _D), jnp.float32),
        pltpu.VMEM_SHARED((_NPAD, _D), jnp.float32),
        pltpu.SemaphoreType.DMA,
        pltpu.SemaphoreType.DMA,
        pltpu.SemaphoreType.DMA,
        pltpu.SemaphoreType.DMA,
    ],
)


_RB = 1000
_GRID = _N // _RB


def _dinv(degp_ref):
    d = degp_ref[...]
    return lax.rsqrt(d[:, 0] + d[:, 1] + 1.0)[:, None]


def _tc1_body(x_ref, w_ref, degp_ref, hs_ref):
    h = jnp.dot(x_ref[...], w_ref[...], preferred_element_type=jnp.float32)
    hs_ref[...] = h * _dinv(degp_ref)


def _tc1(x, W1, degp):
    return pl.pallas_call(
        _tc1_body,
        grid=(_GRID,),
        in_specs=[
            pl.BlockSpec((_RB, _D), lambda i: (i, 0)),
            pl.BlockSpec((_D, _D), lambda i: (0, 0)),
            pl.BlockSpec((_RB, 2), lambda i: (i, 0)),
        ],
        out_specs=pl.BlockSpec((_RB, _D), lambda i: (i, 0)),
        out_shape=jax.ShapeDtypeStruct((_N, _D), jnp.float32),
    )(x, W1, degp)


def _tc2_body(aggp_ref, hs_ref, b_ref, w_ref, degp_ref, out_ref):
    dinv = _dinv(degp_ref)
    a = aggp_ref[...]
    pre = (a[0] + a[1] + hs_ref[...]) * dinv + b_ref[...]
    h1 = jnp.maximum(pre, 0.0)
    out_ref[...] = jnp.dot(h1, w_ref[...],
                           preferred_element_type=jnp.float32) * dinv


def _tc2(aggp, hs, b, W2, degp):
    return pl.pallas_call(
        _tc2_body,
        grid=(_GRID,),
        in_specs=[
            pl.BlockSpec((2, _RB, _D), lambda i: (0, i, 0)),
            pl.BlockSpec((_RB, _D), lambda i: (i, 0)),
            pl.BlockSpec((1, _D), lambda i: (0, 0)),
            pl.BlockSpec((_D, _D), lambda i: (0, 0)),
            pl.BlockSpec((_RB, 2), lambda i: (i, 0)),
        ],
        out_specs=pl.BlockSpec((_RB, _D), lambda i: (i, 0)),
        out_shape=jax.ShapeDtypeStruct((_N, _D), jnp.float32),
    )(aggp, hs, b, W2, degp)


def _tc3_body(aggp_ref, hs_ref, b2_ref, wc1_ref, bc1_ref, wc2_ref, bc2_ref,
              degp_ref, out_ref):
    dinv = _dinv(degp_ref)
    a = aggp_ref[...]
    h2 = jnp.maximum((a[0] + a[1] + hs_ref[...]) * dinv + b2_ref[...], 0.0)
    c = jnp.maximum(
        jnp.dot(h2, wc1_ref[...], preferred_element_type=jnp.float32)
        + bc1_ref[...], 0.0)
    logits = jnp.dot(c, wc2_ref[...],
                     preferred_element_type=jnp.float32) + bc2_ref[...]
    m = jnp.max(logits, axis=1, keepdims=True)
    s = jnp.sum(jnp.exp(logits - m), axis=1, keepdims=True)
    out_ref[...] = logits - m - jnp.log(s)


def _tc3(aggp, hs, b2, Wc1, bc1, Wc2, bc2, degp):
    nh = Wc1.shape[1]
    no = Wc2.shape[1]
    return pl.pallas_call(
        _tc3_body,
        grid=(_GRID,),
        in_specs=[
            pl.BlockSpec((2, _RB, _D), lambda i: (0, i, 0)),
            pl.BlockSpec((_RB, _D), lambda i: (i, 0)),
            pl.BlockSpec((1, _D), lambda i: (0, 0)),
            pl.BlockSpec((_D, nh), lambda i: (0, 0)),
            pl.BlockSpec((1, nh), lambda i: (0, 0)),
            pl.BlockSpec((nh, no), lambda i: (0, 0)),
            pl.BlockSpec((1, no), lambda i: (0, 0)),
            pl.BlockSpec((_RB, 2), lambda i: (i, 0)),
        ],
        out_specs=pl.BlockSpec((_RB, no), lambda i: (i, 0)),
        out_shape=jax.ShapeDtypeStruct((_N, no), jnp.float32),
    )(aggp, hs, b2, Wc1, bc1, Wc2, bc2, degp)


def kernel(x, edge_index, W1, b1, W2, b2, Wc1, bc1, Wc2, bc2):
    ei = edge_index.astype(jnp.int32)
    pad = _EPAD - _E
    src = jnp.concatenate([ei[0], jnp.zeros((pad,), jnp.int32)])
    dst = jnp.concatenate([ei[1], jnp.full((pad,), _N, jnp.int32)])
    src = src.reshape(_NW, _NCH, _CH)
    dst = dst.reshape(_NW, _NCH, _CH)
    ones_ch = jnp.ones((_CH,), jnp.float32)
    zer_deg = jnp.zeros((_RPD,), jnp.float32)
    zer_rows = jnp.zeros((_RPD, _D), jnp.float32)

    degp = _deg_call(dst, ones_ch, zer_deg)[:, :_N].T
    hs1 = _tc1(x, W1, degp)
    aggp1 = _agg_call(hs1, src, dst, zer_rows)[:, :_N]
    hs2 = _tc2(aggp1, hs1, b1.reshape(1, _D), W2, degp)
    aggp2 = _agg_call(hs2, src, dst, zer_rows)[:, :_N]
    return _tc3(aggp2, hs2, b2.reshape(1, _D), Wc1, bc1.reshape(1, -1),
                Wc2, bc2.reshape(1, -1), degp)


# speedup vs baseline: 1.0018x; 1.0018x over previous
import jax
import jax.numpy as jnp
from jax import lax
from jax.experimental import pallas as pl
from jax.experimental.pallas import tpu as pltpu
from jax.experimental.pallas import tpu_sc as plsc

_N = 10000
_E = 320000
_D = 128
_NC = 2
_NS = 16
_NW = _NC * _NS
_CH = 128
_NCH = 80
_EPT = _NCH * _CH
_EPAD = _NW * _EPT
_NPAD = 10240
_RPD = _NPAD // _NS

_mesh = plsc.VectorSubcoreMesh(core_axis_name="c", subcore_axis_name="s")


def _deg_body(dst_hbm, ones_hbm, zeros_hbm, out_hbm, dst_v, ones_v, acc_sh, sem):
    del sem
    cid = lax.axis_index("c")
    sid = lax.axis_index("s")
    wid = sid * _NC + cid
    pltpu.sync_copy(zeros_hbm, acc_sh.at[pl.ds(sid * _RPD, _RPD)])
    pltpu.sync_copy(ones_hbm, ones_v)
    pltpu.sync_copy(dst_hbm.at[wid], dst_v)
    plsc.subcore_barrier()

    def body(j, carry):
        pltpu.sync_copy(ones_v, acc_sh.at[dst_v.at[j]], add=True)
        return carry

    lax.fori_loop(0, _NCH, body, 0)
    plsc.subcore_barrier()
    pltpu.sync_copy(acc_sh.at[pl.ds(sid * _RPD, _RPD)],
                    out_hbm.at[cid, pl.ds(sid * _RPD, _RPD)])


_deg_call = pl.kernel(
    _deg_body,
    out_type=jax.ShapeDtypeStruct((_NC, _NPAD), jnp.float32),
    mesh=_mesh,
    scratch_types=[
        pltpu.VMEM((_NCH, _CH), jnp.int32),
        pltpu.VMEM((_CH,), jnp.float32),
        pltpu.VMEM_SHARED((_NPAD,), jnp.float32),
        pltpu.SemaphoreType.DMA,
    ],
)


def _agg_body(hs_hbm, src_hbm, dst_hbm, zrows_hbm, out_hbm,
              src_v, dring, rows0, rows1, acc_sh, gsem0, gsem1, dsem0, dsem1):
    cid = lax.axis_index("c")
    sid = lax.axis_index("s")
    wid = sid * _NC + cid
    pltpu.sync_copy(src_hbm.at[wid], src_v)
    pltpu.async_copy(dst_hbm.at[wid, 0], dring.at[0], dsem0)
    pltpu.async_copy(dst_hbm.at[wid, 1], dring.at[1], dsem1)
    pltpu.async_copy(hs_hbm.at[src_v.at[0]], rows0, gsem0)
    pltpu.async_copy(hs_hbm.at[src_v.at[1]], rows1, gsem1)
    pltpu.sync_copy(zrows_hbm, acc_sh.at[pl.ds(sid * _RPD, _RPD)])
    plsc.subcore_barrier()

    def body(j2, carry):
        j = 2 * j2
        for b, buf, gsem, dsem in ((0, rows0, gsem0, dsem0),
                                   (1, rows1, gsem1, dsem1)):
            jj = j + b
            pltpu.make_async_copy(hs_hbm.at[src_v.at[jj]], buf, gsem).wait()
            pltpu.make_async_copy(dst_hbm.at[wid, jj], dring.at[b], dsem).wait()
            pltpu.sync_copy(buf, acc_sh.at[dring.at[b]], add=True)

            @pl.when(jj + 2 < _NCH)
            def _():
                pltpu.async_copy(dst_hbm.at[wid, jj + 2], dring.at[b], dsem)
                pltpu.async_copy(hs_hbm.at[src_v.at[jj + 2]], buf, gsem)

        return carry

    lax.fori_loop(0, _NCH // 2, body, 0)
    plsc.subcore_barrier()
    pltpu.sync_copy(acc_sh.at[pl.ds(sid * _RPD, _RPD)],
                    out_hbm.at[cid, pl.ds(sid * _RPD, _RPD)])


_agg_call = pl.kernel(
    _agg_body,
    out_type=jax.ShapeDtypeStruct((_NC, _NPAD, _D), jnp.float32),
    mesh=_mesh,
    scratch_types=[
        pltpu.VMEM((_NCH, _CH), jnp.int32),
        pltpu.VMEM((2, _CH), jnp.int32),
        pltpu.VMEM((_CH, _D), jnp.float32),
        pltpu.VMEM((_CH, _D), jnp.float32),
        pltpu.VMEM_SHARED((_NPAD, _D), jnp.float32),
        pltpu.SemaphoreType.DMA,
        pltpu.SemaphoreType.DMA,
        pltpu.SemaphoreType.DMA,
        pltpu.SemaphoreType.DMA,
    ],
)


_RB = 1000
_GRID = _N // _RB


def _dinv(degp_ref):
    d = degp_ref[...]
    return lax.rsqrt(d[:, 0] + d[:, 1] + 1.0)[:, None]


def _tc1_body(x_ref, w_ref, degp_ref, hs_ref):
    h = jnp.dot(x_ref[...], w_ref[...], preferred_element_type=jnp.float32)
    hs_ref[...] = h * _dinv(degp_ref)


def _tc1(x, W1, degp):
    return pl.pallas_call(
        _tc1_body,
        grid=(_GRID,),
        in_specs=[
            pl.BlockSpec((_RB, _D), lambda i: (i, 0)),
            pl.BlockSpec((_D, _D), lambda i: (0, 0)),
            pl.BlockSpec((_RB, 2), lambda i: (i, 0)),
        ],
        out_specs=pl.BlockSpec((_RB, _D), lambda i: (i, 0)),
        out_shape=jax.ShapeDtypeStruct((_N, _D), jnp.float32),
    )(x, W1, degp)


def _tc2_body(aggp_ref, hs_ref, b_ref, w_ref, degp_ref, out_ref):
    dinv = _dinv(degp_ref)
    a = aggp_ref[...]
    pre = (a[0] + a[1] + hs_ref[...]) * dinv + b_ref[...]
    h1 = jnp.maximum(pre, 0.0)
    out_ref[...] = jnp.dot(h1, w_ref[...],
                           preferred_element_type=jnp.float32) * dinv


def _tc2(aggp, hs, b, W2, degp):
    return pl.pallas_call(
        _tc2_body,
        grid=(_GRID,),
        in_specs=[
            pl.BlockSpec((2, _RB, _D), lambda i: (0, i, 0)),
            pl.BlockSpec((_RB, _D), lambda i: (i, 0)),
            pl.BlockSpec((1, _D), lambda i: (0, 0)),
            pl.BlockSpec((_D, _D), lambda i: (0, 0)),
            pl.BlockSpec((_RB, 2), lambda i: (i, 0)),
        ],
        out_specs=pl.BlockSpec((_RB, _D), lambda i: (i, 0)),
        out_shape=jax.ShapeDtypeStruct((_N, _D), jnp.float32),
    )(aggp, hs, b, W2, degp)


def _tc3_body(aggp_ref, hs_ref, b2_ref, wc1_ref, bc1_ref, wc2_ref, bc2_ref,
              degp_ref, out_ref):
    dinv = _dinv(degp_ref)
    a = aggp_ref[...]
    h2 = jnp.maximum((a[0] + a[1] + hs_ref[...]) * dinv + b2_ref[...], 0.0)
    c = jnp.maximum(
        jnp.dot(h2, wc1_ref[...], preferred_element_type=jnp.float32)
        + bc1_ref[...], 0.0)
    logits = jnp.dot(c, wc2_ref[...],
                     preferred_element_type=jnp.float32) + bc2_ref[...]
    m = jnp.max(logits, axis=1, keepdims=True)
    s = jnp.sum(jnp.exp(logits - m), axis=1, keepdims=True)
    out_ref[...] = logits - m - jnp.log(s)


def _tc3(aggp, hs, b2, Wc1, bc1, Wc2, bc2, degp):
    nh = Wc1.shape[1]
    no = Wc2.shape[1]
    return pl.pallas_call(
        _tc3_body,
        grid=(_GRID,),
        in_specs=[
            pl.BlockSpec((2, _RB, _D), lambda i: (0, i, 0)),
            pl.BlockSpec((_RB, _D), lambda i: (i, 0)),
            pl.BlockSpec((1, _D), lambda i: (0, 0)),
            pl.BlockSpec((_D, nh), lambda i: (0, 0)),
            pl.BlockSpec((1, nh), lambda i: (0, 0)),
            pl.BlockSpec((nh, no), lambda i: (0, 0)),
            pl.BlockSpec((1, no), lambda i: (0, 0)),
            pl.BlockSpec((_RB, 2), lambda i: (i, 0)),
        ],
        out_specs=pl.BlockSpec((_RB, no), lambda i: (i, 0)),
        out_shape=jax.ShapeDtypeStruct((_N, no), jnp.float32),
    )(aggp, hs, b2, Wc1, bc1, Wc2, bc2, degp)


def kernel(x, edge_index, W1, b1, W2, b2, Wc1, bc1, Wc2, bc2):
    ei = edge_index.astype(jnp.int32)
    pad = _EPAD - _E
    src = jnp.concatenate([ei[0], jnp.zeros((pad,), jnp.int32)])
    pad_dst = _N + (jnp.arange(pad, dtype=jnp.int32) % (_NPAD - _N))
    dst = jnp.concatenate([ei[1], pad_dst])
    src = src.reshape(_NW, _NCH, _CH)
    dst = dst.reshape(_NW, _NCH, _CH)
    ones_ch = jnp.ones((_CH,), jnp.float32)
    zer_deg = jnp.zeros((_RPD,), jnp.float32)
    zer_rows = jnp.zeros((_RPD, _D), jnp.float32)

    degp = _deg_call(dst, ones_ch, zer_deg)[:, :_N].T
    hs1 = _tc1(x, W1, degp)
    aggp1 = _agg_call(hs1, src, dst, zer_rows)[:, :_N]
    hs2 = _tc2(aggp1, hs1, b1.reshape(1, _D), W2, degp)
    aggp2 = _agg_call(hs2, src, dst, zer_rows)[:, :_N]
    return _tc3(aggp2, hs2, b2.reshape(1, _D), Wc1, bc1.reshape(1, -1),
                Wc2, bc2.reshape(1, -1), degp)


# speedup vs baseline: 3.2975x; 3.2916x over previous
import jax
import jax.numpy as jnp
from jax import lax
from jax.experimental import pallas as pl
from jax.experimental.pallas import tpu as pltpu
from jax.experimental.pallas import tpu_sc as plsc

_N = 10000
_E = 320000
_D = 128
_NC = 2
_NS = 16
_NW = _NC * _NS
_CH = 128
_NCH = 80
_EPT = _NCH * _CH
_EPAD = _NW * _EPT
_NPAD = 10240
_RPD = _NPAD // _NS

_mesh = plsc.VectorSubcoreMesh(core_axis_name="c", subcore_axis_name="s")


def _deg_body(dst_hbm, ones_hbm, zeros_hbm, out_hbm, dst_v, ones_v, acc_sh, sem):
    del sem
    cid = lax.axis_index("c")
    sid = lax.axis_index("s")
    wid = sid * _NC + cid
    pltpu.sync_copy(zeros_hbm, acc_sh.at[pl.ds(sid * _RPD, _RPD)])
    pltpu.sync_copy(ones_hbm, ones_v)
    pltpu.sync_copy(dst_hbm.at[wid], dst_v)
    plsc.subcore_barrier()

    def body(j, carry):
        pltpu.sync_copy(ones_v, acc_sh.at[dst_v.at[j]], add=True)
        return carry

    lax.fori_loop(0, _NCH, body, 0)
    plsc.subcore_barrier()
    pltpu.sync_copy(acc_sh.at[pl.ds(sid * _RPD, _RPD)],
                    out_hbm.at[cid, pl.ds(sid * _RPD, _RPD)])


_deg_call = pl.kernel(
    _deg_body,
    out_type=jax.ShapeDtypeStruct((_NC, _NPAD), jnp.float32),
    mesh=_mesh,
    scratch_types=[
        pltpu.VMEM((_NCH, _CH), jnp.int32),
        pltpu.VMEM((_CH,), jnp.float32),
        pltpu.VMEM_SHARED((_NPAD,), jnp.float32),
        pltpu.SemaphoreType.DMA,
    ],
)


def _agg_body(hs_hbm, src_hbm, dst_hbm, zrows_hbm, out_hbm,
              src_v, dring, rows0, rows1, acc_sh, gsem0, gsem1, dsem0, dsem1):
    cid = lax.axis_index("c")
    sid = lax.axis_index("s")
    wid = sid * _NC + cid
    pltpu.sync_copy(src_hbm.at[wid], src_v)
    pltpu.async_copy(dst_hbm.at[wid, 0], dring.at[0], dsem0)
    pltpu.async_copy(dst_hbm.at[wid, 1], dring.at[1], dsem1)
    pltpu.async_copy(hs_hbm.at[src_v.at[0]], rows0, gsem0)
    pltpu.async_copy(hs_hbm.at[src_v.at[1]], rows1, gsem1)
    pltpu.sync_copy(zrows_hbm, acc_sh.at[pl.ds(sid * _RPD, _RPD)])
    plsc.subcore_barrier()

    def body(j2, carry):
        j = 2 * j2
        for b, buf, gsem, dsem in ((0, rows0, gsem0, dsem0),
                                   (1, rows1, gsem1, dsem1)):
            jj = j + b
            pltpu.make_async_copy(hs_hbm.at[src_v.at[jj]], buf, gsem).wait()
            pltpu.make_async_copy(dst_hbm.at[wid, jj], dring.at[b], dsem).wait()
            pltpu.sync_copy(buf, acc_sh.at[dring.at[b]], add=True)

            @pl.when(jj + 2 < _NCH)
            def _():
                pltpu.async_copy(dst_hbm.at[wid, jj + 2], dring.at[b], dsem)
                pltpu.async_copy(hs_hbm.at[src_v.at[jj + 2]], buf, gsem)

        return carry

    lax.fori_loop(0, _NCH // 2, body, 0)
    plsc.subcore_barrier()
    pltpu.sync_copy(acc_sh.at[pl.ds(sid * _RPD, _RPD)],
                    out_hbm.at[cid, pl.ds(sid * _RPD, _RPD)])


_agg_call = pl.kernel(
    _agg_body,
    out_type=jax.ShapeDtypeStruct((_NC, _NPAD, _D), jnp.float32),
    mesh=_mesh,
    scratch_types=[
        pltpu.VMEM((_NCH, _CH), jnp.int32),
        pltpu.VMEM((2, _CH), jnp.int32),
        pltpu.VMEM((_CH, _D), jnp.float32),
        pltpu.VMEM((_CH, _D), jnp.float32),
        pltpu.VMEM_SHARED((_NPAD, _D), jnp.float32),
        pltpu.SemaphoreType.DMA,
        pltpu.SemaphoreType.DMA,
        pltpu.SemaphoreType.DMA,
        pltpu.SemaphoreType.DMA,
    ],
)


_RB = 1000
_GRID = _N // _RB


def _dinv(degp_ref):
    d = degp_ref[...]
    return lax.rsqrt(d[:, 0] + d[:, 1] + 1.0)[:, None]


def _tc1_body(x_ref, w_ref, degp_ref, hs_ref):
    h = jnp.dot(x_ref[...], w_ref[...], preferred_element_type=jnp.float32)
    hs_ref[...] = h * _dinv(degp_ref)


def _tc1(x, W1, degp):
    return pl.pallas_call(
        _tc1_body,
        grid=(_GRID,),
        in_specs=[
            pl.BlockSpec((_RB, _D), lambda i: (i, 0)),
            pl.BlockSpec((_D, _D), lambda i: (0, 0)),
            pl.BlockSpec((_RB, 2), lambda i: (i, 0)),
        ],
        out_specs=pl.BlockSpec((_RB, _D), lambda i: (i, 0)),
        out_shape=jax.ShapeDtypeStruct((_N, _D), jnp.float32),
    )(x, W1, degp)


def _tc2_body(aggp_ref, hs_ref, b_ref, w_ref, degp_ref, out_ref):
    dinv = _dinv(degp_ref)
    a = aggp_ref[...]
    pre = (a[0] + a[1] + hs_ref[...]) * dinv + b_ref[...]
    h1 = jnp.maximum(pre, 0.0)
    out_ref[...] = jnp.dot(h1, w_ref[...],
                           preferred_element_type=jnp.float32) * dinv


def _tc2(aggp, hs, b, W2, degp):
    return pl.pallas_call(
        _tc2_body,
        grid=(_GRID,),
        in_specs=[
            pl.BlockSpec((2, _RB, _D), lambda i: (0, i, 0)),
            pl.BlockSpec((_RB, _D), lambda i: (i, 0)),
            pl.BlockSpec((1, _D), lambda i: (0, 0)),
            pl.BlockSpec((_D, _D), lambda i: (0, 0)),
            pl.BlockSpec((_RB, 2), lambda i: (i, 0)),
        ],
        out_specs=pl.BlockSpec((_RB, _D), lambda i: (i, 0)),
        out_shape=jax.ShapeDtypeStruct((_N, _D), jnp.float32),
    )(aggp, hs, b, W2, degp)


def _tc3_body(aggp_ref, hs_ref, b2_ref, wc1_ref, bc1_ref, wc2_ref, bc2_ref,
              degp_ref, out_ref):
    dinv = _dinv(degp_ref)
    a = aggp_ref[...]
    h2 = jnp.maximum((a[0] + a[1] + hs_ref[...]) * dinv + b2_ref[...], 0.0)
    c = jnp.maximum(
        jnp.dot(h2, wc1_ref[...], preferred_element_type=jnp.float32)
        + bc1_ref[...], 0.0)
    logits = jnp.dot(c, wc2_ref[...],
                     preferred_element_type=jnp.float32) + bc2_ref[...]
    m = jnp.max(logits, axis=1, keepdims=True)
    s = jnp.sum(jnp.exp(logits - m), axis=1, keepdims=True)
    out_ref[...] = logits - m - jnp.log(s)


def _tc3(aggp, hs, b2, Wc1, bc1, Wc2, bc2, degp):
    nh = Wc1.shape[1]
    no = Wc2.shape[1]
    return pl.pallas_call(
        _tc3_body,
        grid=(_GRID,),
        in_specs=[
            pl.BlockSpec((2, _RB, _D), lambda i: (0, i, 0)),
            pl.BlockSpec((_RB, _D), lambda i: (i, 0)),
            pl.BlockSpec((1, _D), lambda i: (0, 0)),
            pl.BlockSpec((_D, nh), lambda i: (0, 0)),
            pl.BlockSpec((1, nh), lambda i: (0, 0)),
            pl.BlockSpec((nh, no), lambda i: (0, 0)),
            pl.BlockSpec((1, no), lambda i: (0, 0)),
            pl.BlockSpec((_RB, 2), lambda i: (i, 0)),
        ],
        out_specs=pl.BlockSpec((_RB, no), lambda i: (i, 0)),
        out_shape=jax.ShapeDtypeStruct((_N, no), jnp.float32),
    )(aggp, hs, b2, Wc1, bc1, Wc2, bc2, degp)


def kernel(x, edge_index, W1, b1, W2, b2, Wc1, bc1, Wc2, bc2):
    ei = edge_index.astype(jnp.int32)
    pad = _EPAD - _E
    pad_src = jnp.arange(pad, dtype=jnp.int32) % _N
    src = jnp.concatenate([ei[0], pad_src])
    pad_dst = _N + (jnp.arange(pad, dtype=jnp.int32) % (_NPAD - _N))
    dst = jnp.concatenate([ei[1], pad_dst])
    src = src.reshape(_NW, _NCH, _CH)
    dst = dst.reshape(_NW, _NCH, _CH)
    ones_ch = jnp.ones((_CH,), jnp.float32)
    zer_deg = jnp.zeros((_RPD,), jnp.float32)
    zer_rows = jnp.zeros((_RPD, _D), jnp.float32)

    degp = _deg_call(dst, ones_ch, zer_deg)[:, :_N].T
    hs1 = _tc1(x, W1, degp)
    aggp1 = _agg_call(hs1, src, dst, zer_rows)[:, :_N]
    hs2 = _tc2(aggp1, hs1, b1.reshape(1, _D), W2, degp)
    aggp2 = _agg_call(hs2, src, dst, zer_rows)[:, :_N]
    return _tc3(aggp2, hs2, b2.reshape(1, _D), Wc1, bc1.reshape(1, -1),
                Wc2, bc2.reshape(1, -1), degp)
